# baseline (speedup 1.0000x reference)
import functools

import jax
import jax.numpy as jnp
from jax import lax
from jax.experimental import pallas as pl
from jax.experimental.pallas import tpu as pltpu
from jax.experimental.pallas import tpu_sc as plsc

B = 16384
S = 16
K = S + 1
D = 64
V = 1000000

NC = 2
NS = 16
NW = NC * NS
ROWS_PER_W = B // NW
CHUNK = 16
NCHUNK = ROWS_PER_W // CHUNK
WIDX = CHUNK * K
_IDX_SPLITS = [(0, 128), (128, 128), (256, 16)]

_TBLK = 512


def _tc_retile(wt, eye):
    def body(x_ref, i_ref, o_ref):
        o_ref[:, 0:D] = jax.lax.dot_general(
            x_ref[...], i_ref[...], (((0,), (0,)), ((), ())),
            preferred_element_type=jnp.float32)

    grid = (V + _TBLK - 1) // _TBLK
    return pl.pallas_call(
        body,
        grid=(grid,),
        in_specs=[
            pl.BlockSpec((D, _TBLK), lambda i: (0, i)),
            pl.BlockSpec((D, D), lambda i: (0, 0)),
        ],
        out_specs=pl.BlockSpec((_TBLK, 128), lambda i: (i, 0)),
        out_shape=jax.ShapeDtypeStruct((V, 128), jnp.float32),
    )(wt, eye)


def _sc_scores(doc_ids, word_ids, dpad, wpad):
    mesh = plsc.VectorSubcoreMesh(core_axis_name="c", subcore_axis_name="s")

    @functools.partial(
        pl.kernel,
        mesh=mesh,
        out_type=jax.ShapeDtypeStruct((B * K,), jnp.float32),
        scratch_types=[
            pltpu.VMEM((CHUNK,), jnp.int32),
            pltpu.VMEM((WIDX,), jnp.int32),
            pltpu.VMEM((CHUNK, 128), jnp.float32),
            pltpu.VMEM((WIDX, 128), jnp.float32),
            pltpu.VMEM((WIDX,), jnp.float32),
            pltpu.SemaphoreType.DMA,
            pltpu.SemaphoreType.DMA,
        ],
    )
    def kern(doc_ids_h, word_ids_h, dpad_h, wpad_h, out_h,
             didx, widx, drows, wrows, obuf, dsem, wsem):
        wid = lax.axis_index("s") * NC + lax.axis_index("c")
        base = wid * ROWS_PER_W
        lane = lax.iota(jnp.int32, 16)
        perms = [lane ^ sh for sh in (8, 4, 2, 1)]

        def chunk_body(c, _):
            rb = base + c * CHUNK
            pltpu.sync_copy(doc_ids_h.at[pl.ds(rb, CHUNK)], didx)
            pltpu.sync_copy(word_ids_h.at[pl.ds(rb * K, WIDX)], widx)
            dcp = pltpu.async_copy(dpad_h.at[didx], drows, dsem)
            wcps = [
                pltpu.async_copy(
                    wpad_h.at[widx.at[pl.ds(off, n)]],
                    wrows.at[pl.ds(off, n)], wsem)
                for off, n in _IDX_SPLITS
            ]
            dcp.wait()
            for cp in wcps:
                cp.wait()

            def row_body(r, res):
                dvec = [drows[r, pl.ds(i * 16, 16)] for i in range(4)]
                sel = lane == r
                new = []
                for k in range(K):
                    row = r * K + k
                    acc = dvec[0] * wrows[row, pl.ds(0, 16)]
                    for i in range(1, 4):
                        acc = acc + dvec[i] * wrows[row, pl.ds(i * 16, 16)]
                    for p in perms:
                        acc = acc + jnp.take(acc, p)
                    new.append(jnp.where(sel, acc, res[k]))
                return tuple(new)

            zero = jnp.zeros((16,), jnp.float32)
            res = lax.fori_loop(0, CHUNK, row_body, (zero,) * K)
            obuf[pl.ds(0, 16)] = res[0]
            for k in range(1, K):
                obuf[pl.ds(k * 16, 16)] = -res[k]
            pltpu.sync_copy(obuf, out_h.at[pl.ds(rb * K, WIDX)])
            return 0

        lax.fori_loop(0, NCHUNK, chunk_body, 0)

    return kern(doc_ids, word_ids, dpad, wpad)


def _tc_loss(scores):
    def body(x_ref, o_ref):
        x = x_ref[...]
        ls = jnp.minimum(x, 0.0) - jnp.log1p(jnp.exp(-jnp.abs(x)))
        o_ref[0, 0] = -jnp.sum(ls) / B

    x2 = scores.reshape(B * K // 128, 128)
    out = pl.pallas_call(
        body,
        out_shape=jax.ShapeDtypeStruct((1, 1), jnp.float32),
        out_specs=pl.BlockSpec(memory_space=pltpu.SMEM),
    )(x2)
    return out[0, 0]


def kernel(input_labels, out_labels, num_sampled, word_embed, out_embed, doc_embed):
    batch = input_labels.shape[0]
    num_words = word_embed.shape[0]
    doc_ids = input_labels[:, -1]
    center_ids = input_labels[:, 0]
    nkey = jax.random.key(42)
    _, nk2 = jax.random.split(nkey)
    center_noise = jax.random.randint(nk2, (batch, S), 0, num_words, dtype=jnp.int32)
    word_ids = jnp.concatenate([center_ids[:, None], center_noise], axis=1).reshape(-1)

    eye = jnp.eye(D, dtype=jnp.float32)
    wpad = _tc_retile(word_embed.T, eye)
    dpad = _tc_retile(doc_embed.T, eye)
    scores = _sc_scores(doc_ids, word_ids, dpad, wpad)
    loss = _tc_loss(scores)
    loss = loss + jnp.asarray(num_sampled - num_sampled, dtype=loss.dtype)
    return (loss, jnp.float32(0.0))


# speedup vs baseline: 3.8114x; 3.8114x over previous
import functools

import jax
import jax.numpy as jnp
from jax import lax
from jax.experimental import pallas as pl
from jax.experimental.pallas import tpu as pltpu
from jax.experimental.pallas import tpu_sc as plsc

B = 16384
S = 16
K = S + 1
D = 64
V = 1000000

NC = 2
NS = 16
NW = NC * NS
ROWS_PER_W = B // NW
CHUNK = 16
NCHUNK = ROWS_PER_W // CHUNK
WIDX = CHUNK * K
_IDX_SPLITS = [(0, 128), (128, 128), (256, 16)]

_TBLK = 8192


def _tc_retile(wt, eye):
    def body(x_ref, i_ref, o_ref):
        o_ref[:, 0:D] = jax.lax.dot_general(
            x_ref[...], i_ref[...], (((0,), (0,)), ((), ())),
            preferred_element_type=jnp.float32)

    grid = (V + _TBLK - 1) // _TBLK
    return pl.pallas_call(
        body,
        grid=(grid,),
        in_specs=[
            pl.BlockSpec((D, _TBLK), lambda i: (0, i)),
            pl.BlockSpec((D, D), lambda i: (0, 0)),
        ],
        out_specs=pl.BlockSpec((_TBLK, 128), lambda i: (i, 0)),
        out_shape=jax.ShapeDtypeStruct((V, 128), jnp.float32),
    )(wt, eye)


def _sc_scores(doc_ids, word_ids, dpad, wpad):
    mesh = plsc.VectorSubcoreMesh(core_axis_name="c", subcore_axis_name="s")

    @functools.partial(
        pl.kernel,
        mesh=mesh,
        out_type=jax.ShapeDtypeStruct((B * K,), jnp.float32),
        scratch_types=[
            pltpu.VMEM((CHUNK,), jnp.int32),
            pltpu.VMEM((WIDX,), jnp.int32),
            pltpu.VMEM((CHUNK, 128), jnp.float32),
            pltpu.VMEM((WIDX, 128), jnp.float32),
            pltpu.VMEM((WIDX,), jnp.float32),
            pltpu.SemaphoreType.DMA,
            pltpu.SemaphoreType.DMA,
        ],
    )
    def kern(doc_ids_h, word_ids_h, dpad_h, wpad_h, out_h,
             didx, widx, drows, wrows, obuf, dsem, wsem):
        wid = lax.axis_index("s") * NC + lax.axis_index("c")
        base = wid * ROWS_PER_W
        lane = lax.iota(jnp.int32, 16)
        perms = [lane ^ sh for sh in (8, 4, 2, 1)]

        def chunk_body(c, _):
            rb = base + c * CHUNK
            pltpu.sync_copy(doc_ids_h.at[pl.ds(rb, CHUNK)], didx)
            pltpu.sync_copy(word_ids_h.at[pl.ds(rb * K, WIDX)], widx)
            dcp = pltpu.async_copy(dpad_h.at[didx], drows, dsem)
            wcps = [
                pltpu.async_copy(
                    wpad_h.at[widx.at[pl.ds(off, n)]],
                    wrows.at[pl.ds(off, n)], wsem)
                for off, n in _IDX_SPLITS
            ]
            dcp.wait()
            for cp in wcps:
                cp.wait()

            def row_body(r, res):
                dvec = [drows[r, pl.ds(i * 16, 16)] for i in range(4)]
                sel = lane == r
                new = []
                for k in range(K):
                    row = r * K + k
                    acc = dvec[0] * wrows[row, pl.ds(0, 16)]
                    for i in range(1, 4):
                        acc = acc + dvec[i] * wrows[row, pl.ds(i * 16, 16)]
                    for p in perms:
                        acc = acc + jnp.take(acc, p)
                    new.append(jnp.where(sel, acc, res[k]))
                return tuple(new)

            zero = jnp.zeros((16,), jnp.float32)
            res = lax.fori_loop(0, CHUNK, row_body, (zero,) * K)
            obuf[pl.ds(0, 16)] = res[0]
            for k in range(1, K):
                obuf[pl.ds(k * 16, 16)] = -res[k]
            pltpu.sync_copy(obuf, out_h.at[pl.ds(rb * K, WIDX)])
            return 0

        lax.fori_loop(0, NCHUNK, chunk_body, 0)

    return kern(doc_ids, word_ids, dpad, wpad)


def _tc_loss(scores):
    def body(x_ref, o_ref):
        x = x_ref[...]
        ls = jnp.minimum(x, 0.0) - jnp.log1p(jnp.exp(-jnp.abs(x)))
        o_ref[0, 0] = -jnp.sum(ls) / B

    x2 = scores.reshape(B * K // 128, 128)
    out = pl.pallas_call(
        body,
        out_shape=jax.ShapeDtypeStruct((1, 1), jnp.float32),
        out_specs=pl.BlockSpec(memory_space=pltpu.SMEM),
    )(x2)
    return out[0, 0]


def kernel(input_labels, out_labels, num_sampled, word_embed, out_embed, doc_embed):
    batch = input_labels.shape[0]
    num_words = word_embed.shape[0]
    doc_ids = input_labels[:, -1]
    center_ids = input_labels[:, 0]
    nkey = jax.random.key(42)
    _, nk2 = jax.random.split(nkey)
    center_noise = jax.random.randint(nk2, (batch, S), 0, num_words, dtype=jnp.int32)
    word_ids = jnp.concatenate([center_ids[:, None], center_noise], axis=1).reshape(-1)

    eye = jnp.eye(D, dtype=jnp.float32)
    wpad = _tc_retile(word_embed.T, eye)
    dpad = _tc_retile(doc_embed.T, eye)
    scores = _sc_scores(doc_ids, word_ids, dpad, wpad)
    loss = _tc_loss(scores)
    loss = loss + jnp.asarray(num_sampled - num_sampled, dtype=loss.dtype)
    return (loss, jnp.float32(0.0))


# speedup vs baseline: 4.2080x; 1.1041x over previous
import functools

import jax
import jax.numpy as jnp
from jax import lax
from jax.experimental import pallas as pl
from jax.experimental.pallas import tpu as pltpu
from jax.experimental.pallas import tpu_sc as plsc

B = 16384
S = 16
K = S + 1
D = 64
V = 1000000
VH = 512000

NC = 2
NS = 16
NW = NC * NS
ROWS_PER_W = B // NW
CHUNK = 32
NCHUNK = ROWS_PER_W // CHUNK
WIDX = CHUNK * K
_IDX_SPLITS = [(0, 128), (128, 128), (256, 128), (384, 128), (512, 32)]

_TBLK = 4096
_NBLK = VH // _TBLK


def _tc_retile(wt, eye):
    def body(xl_ref, xr_ref, i_ref, o_ref):
        dn = (((0,), (0,)), ((), ()))
        o_ref[:, 0:D] = jax.lax.dot_general(
            xl_ref[...], i_ref[...], dn, preferred_element_type=jnp.float32)
        o_ref[:, D:128] = jax.lax.dot_general(
            xr_ref[...], i_ref[...], dn, preferred_element_type=jnp.float32)

    packed = pl.pallas_call(
        body,
        grid=(_NBLK,),
        in_specs=[
            pl.BlockSpec((D, _TBLK), lambda i: (0, i)),
            pl.BlockSpec((D, _TBLK),
                         lambda i: (0, jnp.minimum(i + _NBLK, V // _TBLK))),
            pl.BlockSpec((D, D), lambda i: (0, 0)),
        ],
        out_specs=pl.BlockSpec((_TBLK, 128), lambda i: (i, 0)),
        out_shape=jax.ShapeDtypeStruct((VH, 128), jnp.float32),
    )(wt, wt, eye)
    return packed.reshape(2 * VH, D)


def _pack_idx(ids):
    return jnp.where(ids < VH, 2 * ids, 2 * (ids - VH) + 1)


def _sc_scores(doc_ids, word_ids, dtab, wtab):
    mesh = plsc.VectorSubcoreMesh(core_axis_name="c", subcore_axis_name="s")

    @functools.partial(
        pl.kernel,
        mesh=mesh,
        compiler_params=pltpu.CompilerParams(use_tc_tiling_on_sc=False),
        out_type=jax.ShapeDtypeStruct((B * K,), jnp.float32),
        scratch_types=[
            pltpu.VMEM((CHUNK,), jnp.int32),
            pltpu.VMEM((WIDX,), jnp.int32),
            pltpu.VMEM((CHUNK, D), jnp.float32),
            pltpu.VMEM((WIDX, D), jnp.float32),
            pltpu.VMEM((WIDX,), jnp.float32),
            pltpu.SemaphoreType.DMA,
            pltpu.SemaphoreType.DMA,
        ],
    )
    def kern(doc_ids_h, word_ids_h, dtab_h, wtab_h, out_h,
             didx, widx, drows, wrows, obuf, dsem, wsem):
        wid = lax.axis_index("s") * NC + lax.axis_index("c")
        base = wid * ROWS_PER_W
        lane = lax.iota(jnp.int32, 16)
        perms = [lane ^ sh for sh in (8, 4, 2, 1)]

        def chunk_body(c, _):
            rb = base + c * CHUNK
            pltpu.sync_copy(doc_ids_h.at[pl.ds(rb, CHUNK)], didx)
            pltpu.sync_copy(word_ids_h.at[pl.ds(rb * K, WIDX)], widx)
            dcp = pltpu.async_copy(dtab_h.at[didx], drows, dsem)
            wcps = [
                pltpu.async_copy(
                    wtab_h.at[widx.at[pl.ds(off, n)]],
                    wrows.at[pl.ds(off, n)], wsem)
                for off, n in _IDX_SPLITS
            ]
            dcp.wait()
            for cp in wcps:
                cp.wait()

            for g in range(CHUNK // 16):
                def row_body(r, res, g=g):
                    gr = g * 16 + r
                    dvec = [drows[gr, pl.ds(i * 16, 16)] for i in range(4)]
                    sel = lane == r
                    new = []
                    for k in range(K):
                        row = gr * K + k
                        acc = dvec[0] * wrows[row, pl.ds(0, 16)]
                        for i in range(1, 4):
                            acc = acc + dvec[i] * wrows[row, pl.ds(i * 16, 16)]
                        for p in perms:
                            acc = acc + jnp.take(acc, p)
                        new.append(jnp.where(sel, acc, res[k]))
                    return tuple(new)

                zero = jnp.zeros((16,), jnp.float32)
                res = lax.fori_loop(0, 16, row_body, (zero,) * K)
                obuf[pl.ds(g * 16 * K, 16)] = res[0]
                for k in range(1, K):
                    obuf[pl.ds(g * 16 * K + k * 16, 16)] = -res[k]
            pltpu.sync_copy(obuf, out_h.at[pl.ds(rb * K, WIDX)])
            return 0

        lax.fori_loop(0, NCHUNK, chunk_body, 0)

    return kern(doc_ids, word_ids, dtab, wtab)


def _tc_loss(scores):
    def body(x_ref, o_ref):
        x = x_ref[...]
        ls = jnp.minimum(x, 0.0) - jnp.log1p(jnp.exp(-jnp.abs(x)))
        o_ref[0, 0] = -jnp.sum(ls) / B

    x2 = scores.reshape(B * K // 128, 128)
    out = pl.pallas_call(
        body,
        out_shape=jax.ShapeDtypeStruct((1, 1), jnp.float32),
        out_specs=pl.BlockSpec(memory_space=pltpu.SMEM),
    )(x2)
    return out[0, 0]


def kernel(input_labels, out_labels, num_sampled, word_embed, out_embed, doc_embed):
    batch = input_labels.shape[0]
    num_words = word_embed.shape[0]
    doc_ids = input_labels[:, -1]
    center_ids = input_labels[:, 0]
    nkey = jax.random.key(42)
    _, nk2 = jax.random.split(nkey)
    center_noise = jax.random.randint(nk2, (batch, S), 0, num_words, dtype=jnp.int32)
    word_ids = jnp.concatenate([center_ids[:, None], center_noise], axis=1).reshape(-1)

    eye = jnp.eye(D, dtype=jnp.float32)
    wtab = _tc_retile(word_embed.T, eye)
    dtab = _tc_retile(doc_embed.T, eye)
    scores = _sc_scores(_pack_idx(doc_ids), _pack_idx(word_ids), dtab, wtab)
    loss = _tc_loss(scores)
    loss = loss + jnp.asarray(num_sampled - num_sampled, dtype=loss.dtype)
    return (loss, jnp.float32(0.0))


# speedup vs baseline: 4.7309x; 1.1243x over previous
import functools

import jax
import jax.numpy as jnp
from jax import lax
from jax.experimental import pallas as pl
from jax.experimental.pallas import tpu as pltpu
from jax.experimental.pallas import tpu_sc as plsc

B = 16384
S = 16
K = S + 1
D = 64
V = 1000000
VH = 507904

NC = 2
NS = 16
NW = NC * NS
ROWS_PER_W = B // NW
CHUNK = 32
NCHUNK = ROWS_PER_W // CHUNK
WIDX = CHUNK * K
_IDX_SPLITS = [(0, 128), (128, 128), (256, 128), (384, 128), (512, 32)]

_TBLK = 8192
_NBLK = VH // _TBLK


def _tc_retile(wt, eye):
    def body(xl_ref, xr_ref, i_ref, o_ref):
        dn = (((0,), (0,)), ((), ()))
        o_ref[:, 0:D] = jax.lax.dot_general(
            xl_ref[...], i_ref[...], dn,
            preferred_element_type=jnp.float32)
        o_ref[:, D:128] = jax.lax.dot_general(
            xr_ref[...], i_ref[...], dn, preferred_element_type=jnp.float32)

    packed = pl.pallas_call(
        body,
        grid=(_NBLK,),
        in_specs=[
            pl.BlockSpec((D, _TBLK), lambda i: (0, i)),
            pl.BlockSpec((D, _TBLK),
                         lambda i: (0, jnp.minimum(i + _NBLK, V // _TBLK))),
            pl.BlockSpec((D, D), lambda i: (0, 0)),
        ],
        out_specs=pl.BlockSpec((_TBLK, 128), lambda i: (i, 0)),
        out_shape=jax.ShapeDtypeStruct((VH, 128), jnp.float32),
    )(wt, wt, eye)
    return packed.reshape(2 * VH, D)


def _pack_idx(ids):
    return jnp.where(ids < VH, 2 * ids, 2 * (ids - VH) + 1)


def _sc_scores(doc_ids, word_ids, dtab, wtab):
    mesh = plsc.VectorSubcoreMesh(core_axis_name="c", subcore_axis_name="s")

    @functools.partial(
        pl.kernel,
        mesh=mesh,
        compiler_params=pltpu.CompilerParams(use_tc_tiling_on_sc=False),
        out_type=jax.ShapeDtypeStruct((B * K,), jnp.float32),
        scratch_types=[
            pltpu.VMEM((CHUNK,), jnp.int32),
            pltpu.VMEM((WIDX,), jnp.int32),
            pltpu.VMEM((CHUNK, D), jnp.float32),
            pltpu.VMEM((WIDX, D), jnp.float32),
            pltpu.VMEM((WIDX,), jnp.float32),
            pltpu.SemaphoreType.DMA,
            pltpu.SemaphoreType.DMA,
        ],
    )
    def kern(doc_ids_h, word_ids_h, dtab_h, wtab_h, out_h,
             didx, widx, drows, wrows, obuf, dsem, wsem):
        wid = lax.axis_index("s") * NC + lax.axis_index("c")
        base = wid * ROWS_PER_W
        lane = lax.iota(jnp.int32, 16)
        perms = [lane ^ sh for sh in (8, 4, 2, 1)]

        def chunk_body(c, _):
            rb = base + c * CHUNK
            pltpu.sync_copy(doc_ids_h.at[pl.ds(rb, CHUNK)], didx)
            pltpu.sync_copy(word_ids_h.at[pl.ds(rb * K, WIDX)], widx)
            dcp = pltpu.async_copy(dtab_h.at[didx], drows, dsem)
            wcps = [
                pltpu.async_copy(
                    wtab_h.at[widx.at[pl.ds(off, n)]],
                    wrows.at[pl.ds(off, n)], wsem)
                for off, n in _IDX_SPLITS
            ]
            dcp.wait()
            for cp in wcps:
                cp.wait()

            for g in range(CHUNK // 16):
                def row_body(r, res, g=g):
                    gr = g * 16 + r
                    dvec = [drows[gr, pl.ds(i * 16, 16)] for i in range(4)]
                    sel = lane == r
                    new = []
                    for k in range(K):
                        row = gr * K + k
                        acc = dvec[0] * wrows[row, pl.ds(0, 16)]
                        for i in range(1, 4):
                            acc = acc + dvec[i] * wrows[row, pl.ds(i * 16, 16)]
                        for p in perms:
                            acc = acc + jnp.take(acc, p)
                        new.append(jnp.where(sel, acc, res[k]))
                    return tuple(new)

                zero = jnp.zeros((16,), jnp.float32)
                res = lax.fori_loop(0, 16, row_body, (zero,) * K)
                obuf[pl.ds(g * 16 * K, 16)] = res[0]
                for k in range(1, K):
                    obuf[pl.ds(g * 16 * K + k * 16, 16)] = -res[k]
            pltpu.sync_copy(obuf, out_h.at[pl.ds(rb * K, WIDX)])
            return 0

        lax.fori_loop(0, NCHUNK, chunk_body, 0)

    return kern(doc_ids, word_ids, dtab, wtab)


def _tc_loss(scores):
    def body(x_ref, o_ref):
        x = x_ref[...]
        ls = jnp.minimum(x, 0.0) - jnp.log1p(jnp.exp(-jnp.abs(x)))
        o_ref[0, 0] = -jnp.sum(ls) / B

    x2 = scores.reshape(B * K // 128, 128)
    out = pl.pallas_call(
        body,
        out_shape=jax.ShapeDtypeStruct((1, 1), jnp.float32),
        out_specs=pl.BlockSpec(memory_space=pltpu.SMEM),
    )(x2)
    return out[0, 0]


def kernel(input_labels, out_labels, num_sampled, word_embed, out_embed, doc_embed):
    batch = input_labels.shape[0]
    num_words = word_embed.shape[0]
    doc_ids = input_labels[:, -1]
    center_ids = input_labels[:, 0]
    nkey = jax.random.key(42)
    _, nk2 = jax.random.split(nkey)
    center_noise = jax.random.randint(nk2, (batch, S), 0, num_words, dtype=jnp.int32)
    word_ids = jnp.concatenate([center_ids[:, None], center_noise], axis=1).reshape(-1)

    eye = jnp.eye(D, dtype=jnp.float32)
    wtab = _tc_retile(word_embed.T, eye)
    dtab = _tc_retile(doc_embed.T, eye)
    scores = _sc_scores(_pack_idx(doc_ids), _pack_idx(word_ids), dtab, wtab)
    loss = _tc_loss(scores)
    loss = loss + jnp.asarray(num_sampled - num_sampled, dtype=loss.dtype)
    return (loss, jnp.float32(0.0))


# speedup vs baseline: 4.9958x; 1.0560x over previous
import functools

import jax
import jax.numpy as jnp
from jax import lax
from jax.experimental import pallas as pl
from jax.experimental.pallas import tpu as pltpu
from jax.experimental.pallas import tpu_sc as plsc

B = 16384
S = 16
K = S + 1
D = 64
V = 1000000
VH = 507904

NC = 2
NS = 16
NW = NC * NS
ROWS_PER_W = B // NW
CHUNK = 32
NCHUNK = ROWS_PER_W // CHUNK
WIDX = CHUNK * K
_IDX_SPLITS = [(0, 128), (128, 128), (256, 128), (384, 128), (512, 32)]

_TBLK = 16384
_NBLK = VH // _TBLK


def _tc_retile(wt, eye):
    def body(xl_ref, xr_ref, i_ref, o_ref):
        dn = (((0,), (0,)), ((), ()))
        o_ref[:, 0:D] = jax.lax.dot_general(
            xl_ref[...], i_ref[...], dn,
            preferred_element_type=jnp.float32)
        o_ref[:, D:128] = jax.lax.dot_general(
            xr_ref[...], i_ref[...], dn, preferred_element_type=jnp.float32)

    packed = pl.pallas_call(
        body,
        grid=(_NBLK,),
        in_specs=[
            pl.BlockSpec((D, _TBLK), lambda i: (0, i)),
            pl.BlockSpec((D, _TBLK),
                         lambda i: (0, jnp.minimum(i + _NBLK, V // _TBLK))),
            pl.BlockSpec((D, D), lambda i: (0, 0)),
        ],
        out_specs=pl.BlockSpec((_TBLK, 128), lambda i: (i, 0)),
        out_shape=jax.ShapeDtypeStruct((VH, 128), jnp.float32),
    )(wt, wt, eye)
    return packed.reshape(2 * VH, D)


def _pack_idx(ids):
    return jnp.where(ids < VH, 2 * ids, 2 * (ids - VH) + 1)


def _sc_scores(doc_ids, word_ids, dtab, wtab):
    mesh = plsc.VectorSubcoreMesh(core_axis_name="c", subcore_axis_name="s")

    @functools.partial(
        pl.kernel,
        mesh=mesh,
        compiler_params=pltpu.CompilerParams(use_tc_tiling_on_sc=False),
        out_type=jax.ShapeDtypeStruct((B * K,), jnp.float32),
        scratch_types=[
            pltpu.VMEM((CHUNK,), jnp.int32),
            pltpu.VMEM((WIDX,), jnp.int32),
            pltpu.VMEM((CHUNK, D), jnp.float32),
            pltpu.VMEM((WIDX, D), jnp.float32),
            pltpu.VMEM((WIDX,), jnp.float32),
            pltpu.SemaphoreType.DMA,
            pltpu.SemaphoreType.DMA,
        ],
    )
    def kern(doc_ids_h, word_ids_h, dtab_h, wtab_h, out_h,
             didx, widx, drows, wrows, obuf, dsem, wsem):
        wid = lax.axis_index("s") * NC + lax.axis_index("c")
        base = wid * ROWS_PER_W
        lane = lax.iota(jnp.int32, 16)
        perms = [lane ^ sh for sh in (8, 4, 2, 1)]

        def chunk_body(c, _):
            rb = base + c * CHUNK
            pltpu.sync_copy(doc_ids_h.at[pl.ds(rb, CHUNK)], didx)
            pltpu.sync_copy(word_ids_h.at[pl.ds(rb * K, WIDX)], widx)
            dcp = pltpu.async_copy(dtab_h.at[didx], drows, dsem)
            wcps = [
                pltpu.async_copy(
                    wtab_h.at[widx.at[pl.ds(off, n)]],
                    wrows.at[pl.ds(off, n)], wsem)
                for off, n in _IDX_SPLITS
            ]
            dcp.wait()
            for cp in wcps:
                cp.wait()

            for g in range(CHUNK // 16):
                def row_body(r, res, g=g):
                    gr = g * 16 + r
                    dvec = [drows[gr, pl.ds(i * 16, 16)] for i in range(4)]
                    sel = lane == r
                    new = []
                    for k in range(K):
                        row = gr * K + k
                        acc = dvec[0] * wrows[row, pl.ds(0, 16)]
                        for i in range(1, 4):
                            acc = acc + dvec[i] * wrows[row, pl.ds(i * 16, 16)]
                        for p in perms:
                            acc = acc + jnp.take(acc, p)
                        new.append(jnp.where(sel, acc, res[k]))
                    return tuple(new)

                zero = jnp.zeros((16,), jnp.float32)
                res = lax.fori_loop(0, 16, row_body, (zero,) * K)
                obuf[pl.ds(g * 16 * K, 16)] = res[0]
                for k in range(1, K):
                    obuf[pl.ds(g * 16 * K + k * 16, 16)] = -res[k]
            pltpu.sync_copy(obuf, out_h.at[pl.ds(rb * K, WIDX)])
            return 0

        lax.fori_loop(0, NCHUNK, chunk_body, 0)

    return kern(doc_ids, word_ids, dtab, wtab)


def _tc_loss(scores):
    def body(x_ref, o_ref):
        x = x_ref[...]
        ls = jnp.minimum(x, 0.0) - jnp.log1p(jnp.exp(-jnp.abs(x)))
        o_ref[0, 0] = -jnp.sum(ls) / B

    x2 = scores.reshape(B * K // 128, 128)
    out = pl.pallas_call(
        body,
        out_shape=jax.ShapeDtypeStruct((1, 1), jnp.float32),
        out_specs=pl.BlockSpec(memory_space=pltpu.SMEM),
    )(x2)
    return out[0, 0]


def kernel(input_labels, out_labels, num_sampled, word_embed, out_embed, doc_embed):
    batch = input_labels.shape[0]
    num_words = word_embed.shape[0]
    doc_ids = input_labels[:, -1]
    center_ids = input_labels[:, 0]
    nkey = jax.random.key(42)
    _, nk2 = jax.random.split(nkey)
    center_noise = jax.random.randint(nk2, (batch, S), 0, num_words, dtype=jnp.int32)
    word_ids = jnp.concatenate([center_ids[:, None], center_noise], axis=1).reshape(-1)

    eye = jnp.eye(D, dtype=jnp.float32)
    wtab = _tc_retile(word_embed.T, eye)
    dtab = _tc_retile(doc_embed.T, eye)
    scores = _sc_scores(_pack_idx(doc_ids), _pack_idx(word_ids), dtab, wtab)
    loss = _tc_loss(scores)
    loss = loss + jnp.asarray(num_sampled - num_sampled, dtype=loss.dtype)
    return (loss, jnp.float32(0.0))


# speedup vs baseline: 5.0753x; 1.0159x over previous
import functools

import jax
import jax.numpy as jnp
from jax import lax
from jax.experimental import pallas as pl
from jax.experimental.pallas import tpu as pltpu
from jax.experimental.pallas import tpu_sc as plsc

B = 16384
S = 16
K = S + 1
D = 64
V = 1000000
VH = 507904

NC = 2
NS = 16
NW = NC * NS
ROWS_PER_W = B // NW
CHUNK = 64
NCHUNK = ROWS_PER_W // CHUNK
WIDX = CHUNK * K
_IDX_SPLITS = [(i * 128, 128) for i in range(WIDX // 128)] + (
    [(WIDX - WIDX % 128, WIDX % 128)] if WIDX % 128 else [])

_TBLK = 16384
_NBLK = VH // _TBLK


def _tc_retile(wt, eye):
    def body(xl_ref, xr_ref, i_ref, o_ref):
        dn = (((0,), (0,)), ((), ()))
        o_ref[:, 0:D] = jax.lax.dot_general(
            xl_ref[...], i_ref[...], dn,
            preferred_element_type=jnp.float32)
        o_ref[:, D:128] = jax.lax.dot_general(
            xr_ref[...], i_ref[...], dn, preferred_element_type=jnp.float32)

    packed = pl.pallas_call(
        body,
        grid=(_NBLK,),
        in_specs=[
            pl.BlockSpec((D, _TBLK), lambda i: (0, i)),
            pl.BlockSpec((D, _TBLK),
                         lambda i: (0, jnp.minimum(i + _NBLK, V // _TBLK))),
            pl.BlockSpec((D, D), lambda i: (0, 0)),
        ],
        out_specs=pl.BlockSpec((_TBLK, 128), lambda i: (i, 0)),
        out_shape=jax.ShapeDtypeStruct((VH, 128), jnp.float32),
    )(wt, wt, eye)
    return packed.reshape(2 * VH, D)


def _pack_idx(ids):
    return jnp.where(ids < VH, 2 * ids, 2 * (ids - VH) + 1)


def _sc_scores(doc_ids, word_ids, dtab, wtab):
    mesh = plsc.VectorSubcoreMesh(core_axis_name="c", subcore_axis_name="s")

    @functools.partial(
        pl.kernel,
        mesh=mesh,
        compiler_params=pltpu.CompilerParams(use_tc_tiling_on_sc=False),
        out_type=jax.ShapeDtypeStruct((B * K,), jnp.float32),
        scratch_types=[
            pltpu.VMEM((CHUNK,), jnp.int32),
            pltpu.VMEM((WIDX,), jnp.int32),
            pltpu.VMEM((CHUNK, D), jnp.float32),
            pltpu.VMEM((WIDX, D), jnp.float32),
            pltpu.VMEM((WIDX,), jnp.float32),
            pltpu.SemaphoreType.DMA,
            pltpu.SemaphoreType.DMA,
        ],
    )
    def kern(doc_ids_h, word_ids_h, dtab_h, wtab_h, out_h,
             didx, widx, drows, wrows, obuf, dsem, wsem):
        wid = lax.axis_index("s") * NC + lax.axis_index("c")
        base = wid * ROWS_PER_W
        lane = lax.iota(jnp.int32, 16)
        perms = [lane ^ sh for sh in (8, 4, 2, 1)]

        def chunk_body(c, _):
            rb = base + c * CHUNK
            pltpu.sync_copy(doc_ids_h.at[pl.ds(rb, CHUNK)], didx)
            pltpu.sync_copy(word_ids_h.at[pl.ds(rb * K, WIDX)], widx)
            dcp = pltpu.async_copy(dtab_h.at[didx], drows, dsem)
            wcps = [
                pltpu.async_copy(
                    wtab_h.at[widx.at[pl.ds(off, n)]],
                    wrows.at[pl.ds(off, n)], wsem)
                for off, n in _IDX_SPLITS
            ]
            dcp.wait()
            for cp in wcps:
                cp.wait()

            for g in range(CHUNK // 16):
                def row_body(r, res, g=g):
                    gr = g * 16 + r
                    dvec = [drows[gr, pl.ds(i * 16, 16)] for i in range(4)]
                    sel = lane == r
                    new = []
                    for k in range(K):
                        row = gr * K + k
                        acc = dvec[0] * wrows[row, pl.ds(0, 16)]
                        for i in range(1, 4):
                            acc = acc + dvec[i] * wrows[row, pl.ds(i * 16, 16)]
                        for p in perms:
                            acc = acc + jnp.take(acc, p)
                        new.append(jnp.where(sel, acc, res[k]))
                    return tuple(new)

                zero = jnp.zeros((16,), jnp.float32)
                res = lax.fori_loop(0, 16, row_body, (zero,) * K)
                obuf[pl.ds(g * 16 * K, 16)] = res[0]
                for k in range(1, K):
                    obuf[pl.ds(g * 16 * K + k * 16, 16)] = -res[k]
            pltpu.sync_copy(obuf, out_h.at[pl.ds(rb * K, WIDX)])
            return 0

        lax.fori_loop(0, NCHUNK, chunk_body, 0)

    return kern(doc_ids, word_ids, dtab, wtab)


def _tc_loss(scores):
    def body(x_ref, o_ref):
        x = x_ref[...]
        ls = jnp.minimum(x, 0.0) - jnp.log1p(jnp.exp(-jnp.abs(x)))
        o_ref[0, 0] = -jnp.sum(ls) / B

    x2 = scores.reshape(B * K // 128, 128)
    out = pl.pallas_call(
        body,
        out_shape=jax.ShapeDtypeStruct((1, 1), jnp.float32),
        out_specs=pl.BlockSpec(memory_space=pltpu.SMEM),
    )(x2)
    return out[0, 0]


def kernel(input_labels, out_labels, num_sampled, word_embed, out_embed, doc_embed):
    batch = input_labels.shape[0]
    num_words = word_embed.shape[0]
    doc_ids = input_labels[:, -1]
    center_ids = input_labels[:, 0]
    nkey = jax.random.key(42)
    _, nk2 = jax.random.split(nkey)
    center_noise = jax.random.randint(nk2, (batch, S), 0, num_words, dtype=jnp.int32)
    word_ids = jnp.concatenate([center_ids[:, None], center_noise], axis=1).reshape(-1)

    eye = jnp.eye(D, dtype=jnp.float32)
    wtab = _tc_retile(word_embed.T, eye)
    dtab = _tc_retile(doc_embed.T, eye)
    scores = _sc_scores(_pack_idx(doc_ids), _pack_idx(word_ids), dtab, wtab)
    loss = _tc_loss(scores)
    loss = loss + jnp.asarray(num_sampled - num_sampled, dtype=loss.dtype)
    return (loss, jnp.float32(0.0))


# speedup vs baseline: 5.0907x; 1.0030x over previous
import functools

import jax
import jax.numpy as jnp
from jax import lax
from jax.experimental import pallas as pl
from jax.experimental.pallas import tpu as pltpu
from jax.experimental.pallas import tpu_sc as plsc

B = 16384
S = 16
K = S + 1
D = 64
V = 1000000
VH = 507904

NC = 2
NS = 16
NW = NC * NS
ROWS_PER_W = B // NW
CHUNK = 64
NCHUNK = ROWS_PER_W // CHUNK
WIDX = CHUNK * K
_IDX_SPLITS = [(i * 128, 128) for i in range(WIDX // 128)] + (
    [(WIDX - WIDX % 128, WIDX % 128)] if WIDX % 128 else [])

_TBLK = 16384
_NBLK = VH // _TBLK


def _tc_retile(wt, eye):
    def body(xl_ref, xr_ref, i_ref, o_ref):
        o_ref[:, 0:D] = xl_ref[...].T
        o_ref[:, D:128] = xr_ref[...].T

    packed = pl.pallas_call(
        body,
        grid=(_NBLK,),
        in_specs=[
            pl.BlockSpec((D, _TBLK), lambda i: (0, i)),
            pl.BlockSpec((D, _TBLK),
                         lambda i: (0, jnp.minimum(i + _NBLK, V // _TBLK))),
            pl.BlockSpec((D, D), lambda i: (0, 0)),
        ],
        out_specs=pl.BlockSpec((_TBLK, 128), lambda i: (i, 0)),
        out_shape=jax.ShapeDtypeStruct((VH, 128), jnp.float32),
    )(wt, wt, eye)
    return packed.reshape(2 * VH, D)


def _pack_idx(ids):
    return jnp.where(ids < VH, 2 * ids, 2 * (ids - VH) + 1)


def _sc_scores(doc_ids, word_ids, dtab, wtab):
    mesh = plsc.VectorSubcoreMesh(core_axis_name="c", subcore_axis_name="s")

    @functools.partial(
        pl.kernel,
        mesh=mesh,
        compiler_params=pltpu.CompilerParams(use_tc_tiling_on_sc=False),
        out_type=jax.ShapeDtypeStruct((B * K,), jnp.float32),
        scratch_types=[
            pltpu.VMEM((CHUNK,), jnp.int32),
            pltpu.VMEM((WIDX,), jnp.int32),
            pltpu.VMEM((CHUNK, D), jnp.float32),
            pltpu.VMEM((WIDX, D), jnp.float32),
            pltpu.VMEM((WIDX,), jnp.float32),
            pltpu.SemaphoreType.DMA,
            pltpu.SemaphoreType.DMA,
        ],
    )
    def kern(doc_ids_h, word_ids_h, dtab_h, wtab_h, out_h,
             didx, widx, drows, wrows, obuf, dsem, wsem):
        wid = lax.axis_index("s") * NC + lax.axis_index("c")
        base = wid * ROWS_PER_W
        lane = lax.iota(jnp.int32, 16)
        perms = [lane ^ sh for sh in (8, 4, 2, 1)]

        def chunk_body(c, _):
            rb = base + c * CHUNK
            pltpu.sync_copy(doc_ids_h.at[pl.ds(rb, CHUNK)], didx)
            pltpu.sync_copy(word_ids_h.at[pl.ds(rb * K, WIDX)], widx)
            dcp = pltpu.async_copy(dtab_h.at[didx], drows, dsem)
            wcps = [
                pltpu.async_copy(
                    wtab_h.at[widx.at[pl.ds(off, n)]],
                    wrows.at[pl.ds(off, n)], wsem)
                for off, n in _IDX_SPLITS
            ]
            dcp.wait()
            for cp in wcps:
                cp.wait()

            for g in range(CHUNK // 16):
                def row_body(r, res, g=g):
                    gr = g * 16 + r
                    dvec = [drows[gr, pl.ds(i * 16, 16)] for i in range(4)]
                    sel = lane == r
                    new = []
                    for k in range(K):
                        row = gr * K + k
                        acc = dvec[0] * wrows[row, pl.ds(0, 16)]
                        for i in range(1, 4):
                            acc = acc + dvec[i] * wrows[row, pl.ds(i * 16, 16)]
                        for p in perms:
                            acc = acc + jnp.take(acc, p)
                        new.append(jnp.where(sel, acc, res[k]))
                    return tuple(new)

                zero = jnp.zeros((16,), jnp.float32)
                res = lax.fori_loop(0, 16, row_body, (zero,) * K)
                obuf[pl.ds(g * 16 * K, 16)] = res[0]
                for k in range(1, K):
                    obuf[pl.ds(g * 16 * K + k * 16, 16)] = -res[k]
            pltpu.sync_copy(obuf, out_h.at[pl.ds(rb * K, WIDX)])
            return 0

        lax.fori_loop(0, NCHUNK, chunk_body, 0)

    return kern(doc_ids, word_ids, dtab, wtab)


def _tc_loss(scores):
    def body(x_ref, o_ref):
        x = x_ref[...]
        ls = jnp.minimum(x, 0.0) - jnp.log1p(jnp.exp(-jnp.abs(x)))
        o_ref[0, 0] = -jnp.sum(ls) / B

    x2 = scores.reshape(B * K // 128, 128)
    out = pl.pallas_call(
        body,
        out_shape=jax.ShapeDtypeStruct((1, 1), jnp.float32),
        out_specs=pl.BlockSpec(memory_space=pltpu.SMEM),
    )(x2)
    return out[0, 0]


def kernel(input_labels, out_labels, num_sampled, word_embed, out_embed, doc_embed):
    batch = input_labels.shape[0]
    num_words = word_embed.shape[0]
    doc_ids = input_labels[:, -1]
    center_ids = input_labels[:, 0]
    nkey = jax.random.key(42)
    _, nk2 = jax.random.split(nkey)
    center_noise = jax.random.randint(nk2, (batch, S), 0, num_words, dtype=jnp.int32)
    word_ids = jnp.concatenate([center_ids[:, None], center_noise], axis=1).reshape(-1)

    eye = jnp.eye(D, dtype=jnp.float32)
    wtab = _tc_retile(word_embed.T, eye)
    dtab = _tc_retile(doc_embed.T, eye)
    scores = _sc_scores(_pack_idx(doc_ids), _pack_idx(word_ids), dtab, wtab)
    loss = _tc_loss(scores)
    loss = loss + jnp.asarray(num_sampled - num_sampled, dtype=loss.dtype)
    return (loss, jnp.float32(0.0))


# speedup vs baseline: 5.1631x; 1.0142x over previous
import functools

import jax
import jax.numpy as jnp
from jax import lax
from jax.experimental import pallas as pl
from jax.experimental.pallas import tpu as pltpu
from jax.experimental.pallas import tpu_sc as plsc

B = 16384
S = 16
K = S + 1
D = 64
V = 1000000
VH = 507904

NC = 2
NS = 16
NW = NC * NS
ROWS_PER_W = B // NW
CHUNK = 64
NCHUNK = ROWS_PER_W // CHUNK
WIDX = CHUNK * K
_IDX_SPLITS = [(i * 128, 128) for i in range(WIDX // 128)] + (
    [(WIDX - WIDX % 128, WIDX % 128)] if WIDX % 128 else [])

_TBLK = 8192
_NBLK = VH // _TBLK


def _tc_retile(wt, dt):
    def body(wl_ref, wr_ref, dl_ref, dr_ref, wo_ref, do_ref):
        wo_ref[:, 0:D] = wl_ref[...].T
        wo_ref[:, D:128] = wr_ref[...].T
        do_ref[:, 0:D] = dl_ref[...].T
        do_ref[:, D:128] = dr_ref[...].T

    left = pl.BlockSpec((D, _TBLK), lambda i: (0, i))
    right = pl.BlockSpec((D, _TBLK),
                         lambda i: (0, jnp.minimum(i + _NBLK, V // _TBLK)))
    out = pl.BlockSpec((_TBLK, 128), lambda i: (i, 0))
    oshape = jax.ShapeDtypeStruct((VH, 128), jnp.float32)
    wp, dp = pl.pallas_call(
        body,
        grid=(_NBLK,),
        in_specs=[left, right, left, right],
        out_specs=[out, out],
        out_shape=[oshape, oshape],
    )(wt, wt, dt, dt)
    return wp.reshape(2 * VH, D), dp.reshape(2 * VH, D)


def _pack_idx(ids):
    return jnp.where(ids < VH, 2 * ids, 2 * (ids - VH) + 1)


def _sc_scores(doc_ids, word_ids, dtab, wtab):
    mesh = plsc.VectorSubcoreMesh(core_axis_name="c", subcore_axis_name="s")

    @functools.partial(
        pl.kernel,
        mesh=mesh,
        compiler_params=pltpu.CompilerParams(use_tc_tiling_on_sc=False),
        out_type=jax.ShapeDtypeStruct((B * K,), jnp.float32),
        scratch_types=[
            pltpu.VMEM((CHUNK,), jnp.int32),
            pltpu.VMEM((WIDX,), jnp.int32),
            pltpu.VMEM((CHUNK, D), jnp.float32),
            pltpu.VMEM((WIDX, D), jnp.float32),
            pltpu.VMEM((WIDX,), jnp.float32),
            pltpu.SemaphoreType.DMA,
            pltpu.SemaphoreType.DMA,
        ],
    )
    def kern(doc_ids_h, word_ids_h, dtab_h, wtab_h, out_h,
             didx, widx, drows, wrows, obuf, dsem, wsem):
        wid = lax.axis_index("s") * NC + lax.axis_index("c")
        base = wid * ROWS_PER_W
        lane = lax.iota(jnp.int32, 16)
        perms = [lane ^ sh for sh in (8, 4, 2, 1)]

        def chunk_body(c, _):
            rb = base + c * CHUNK
            pltpu.sync_copy(doc_ids_h.at[pl.ds(rb, CHUNK)], didx)
            pltpu.sync_copy(word_ids_h.at[pl.ds(rb * K, WIDX)], widx)
            dcp = pltpu.async_copy(dtab_h.at[didx], drows, dsem)
            wcps = [
                pltpu.async_copy(
                    wtab_h.at[widx.at[pl.ds(off, n)]],
                    wrows.at[pl.ds(off, n)], wsem)
                for off, n in _IDX_SPLITS
            ]
            dcp.wait()
            for cp in wcps:
                cp.wait()

            for g in range(CHUNK // 16):
                def row_body(r, res, g=g):
                    gr = g * 16 + r
                    dvec = [drows[gr, pl.ds(i * 16, 16)] for i in range(4)]
                    sel = lane == r
                    new = []
                    for k in range(K):
                        row = gr * K + k
                        acc = dvec[0] * wrows[row, pl.ds(0, 16)]
                        for i in range(1, 4):
                            acc = acc + dvec[i] * wrows[row, pl.ds(i * 16, 16)]
                        for p in perms:
                            acc = acc + jnp.take(acc, p)
                        new.append(jnp.where(sel, acc, res[k]))
                    return tuple(new)

                zero = jnp.zeros((16,), jnp.float32)
                res = lax.fori_loop(0, 16, row_body, (zero,) * K)
                obuf[pl.ds(g * 16 * K, 16)] = res[0]
                for k in range(1, K):
                    obuf[pl.ds(g * 16 * K + k * 16, 16)] = -res[k]
            pltpu.sync_copy(obuf, out_h.at[pl.ds(rb * K, WIDX)])
            return 0

        lax.fori_loop(0, NCHUNK, chunk_body, 0)

    return kern(doc_ids, word_ids, dtab, wtab)


def _tc_loss(scores):
    def body(x_ref, o_ref):
        x = x_ref[...]
        ls = jnp.minimum(x, 0.0) - jnp.log1p(jnp.exp(-jnp.abs(x)))
        o_ref[0, 0] = -jnp.sum(ls) / B

    x2 = scores.reshape(B * K // 128, 128)
    out = pl.pallas_call(
        body,
        out_shape=jax.ShapeDtypeStruct((1, 1), jnp.float32),
        out_specs=pl.BlockSpec(memory_space=pltpu.SMEM),
    )(x2)
    return out[0, 0]


def kernel(input_labels, out_labels, num_sampled, word_embed, out_embed, doc_embed):
    batch = input_labels.shape[0]
    num_words = word_embed.shape[0]
    doc_ids = input_labels[:, -1]
    center_ids = input_labels[:, 0]
    nkey = jax.random.key(42)
    _, nk2 = jax.random.split(nkey)
    center_noise = jax.random.randint(nk2, (batch, S), 0, num_words, dtype=jnp.int32)
    word_ids = jnp.concatenate([center_ids[:, None], center_noise], axis=1).reshape(-1)

    wtab, dtab = _tc_retile(word_embed.T, doc_embed.T)
    scores = _sc_scores(_pack_idx(doc_ids), _pack_idx(word_ids), dtab, wtab)
    loss = _tc_loss(scores)
    loss = loss + jnp.asarray(num_sampled - num_sampled, dtype=loss.dtype)
    return (loss, jnp.float32(0.0))
